# baseline (speedup 1.0000x reference)
import functools

import jax
import jax.numpy as jnp
from jax import lax
from jax.experimental import pallas as pl
from jax.experimental.pallas import tpu as pltpu
from jax.experimental.pallas import tpu_sc as plsc

N = 10000
E = 160000
R = 16
D = 128

BN = 1000
NB = N // BN

NC = 2
NS = 16
NW = NC * NS
CH = 128
G = E // (2 * CH)
BASE_W, EXTRA_W = G // NW, G % NW
BASE_S, EXTRA_S = G // NS, G % NS
RB = 40
NRB = N // RB
BASE_R, EXTRA_R = NRB // NS, NRB % NS
NCH = E // CH
BASE_C, EXTRA_C = NCH // NW, NCH % NW
CZ = 1280
NCZ = (N * R) // CZ

_mesh = plsc.VectorSubcoreMesh(core_axis_name="c", subcore_axis_name="s")

_GDN = lax.GatherDimensionNumbers(
    offset_dims=(), collapsed_slice_dims=(0,), start_index_map=(0,)
)


def _lane_bcast(vec16, lane):
    idx = jnp.full((16,), lane, jnp.int32)
    return lax.gather(vec16, idx[:, None], _GDN, (1,),
                      mode=lax.GatherScatterMode.PROMISE_IN_BOUNDS)


@functools.partial(
    pl.kernel,
    out_type=jax.ShapeDtypeStruct((N * R,), jnp.float32),
    mesh=_mesh,
    scratch_types=[
        pltpu.VMEM((2, CH), jnp.int32),
        pltpu.VMEM((CH,), jnp.float32),
        pltpu.VMEM((CZ,), jnp.float32),
        pltpu.VMEM_SHARED((N * R,), jnp.float32),
    ],
)
def _sc_counts(comb_ref, cnt_ref, cb2, ones, zbuf, cnt_sh):
    c = lax.axis_index("c")
    s = lax.axis_index("s")

    z16 = jnp.zeros((16,), jnp.float32)
    o16 = jnp.ones((16,), jnp.float32)

    def zfill(i, carry):
        zbuf[pl.ds(i * 16, 16)] = z16
        return carry

    lax.fori_loop(0, CZ // 16, zfill, 0)
    for k in range(CH // 16):
        ones[pl.ds(k * 16, 16)] = o16

    @pl.when(c == 0)
    def _():
        nz = jnp.where(s < NCZ % NS, NCZ // NS + 1, NCZ // NS)

        def zloop(t, carry):
            b = s + NS * t
            pltpu.sync_copy(zbuf, cnt_sh.at[pl.ds(b * CZ, CZ)])
            return carry

        lax.fori_loop(0, nz, zloop, 0)

    plsc.subcore_barrier()

    @pl.when(c == 0)
    def _():
        nb = jnp.where(s < EXTRA_S, BASE_S + 1, BASE_S)

        def bloop(t, carry):
            g = s + NS * t
            pltpu.sync_copy(comb_ref.at[g], cb2)
            for j in range(2):
                pltpu.sync_copy(ones, cnt_sh.at[cb2.at[j]], add=True)
            return carry

        lax.fori_loop(0, nb, bloop, 0)

    plsc.subcore_barrier()

    @pl.when(c == 0)
    def _():
        nz = jnp.where(s < NCZ % NS, NCZ // NS + 1, NCZ // NS)

        def wloop(t, carry):
            b = s + NS * t
            pltpu.sync_copy(cnt_sh.at[pl.ds(b * CZ, CZ)],
                            cnt_ref.at[pl.ds(b * CZ, CZ)])
            return carry

        lax.fori_loop(0, nz, wloop, 0)


@functools.partial(
    pl.kernel,
    out_type=jax.ShapeDtypeStruct((NC * N, D), jnp.float32),
    mesh=_mesh,
    scratch_types=[
        pltpu.VMEM((1, CH), jnp.int32),
        pltpu.VMEM((1, CH), jnp.int32),
        pltpu.VMEM((1, CH), jnp.int32),
        pltpu.VMEM((1, CH), jnp.float32),
        pltpu.VMEM((1, CH), jnp.int32),
        pltpu.VMEM((1, CH), jnp.int32),
        pltpu.VMEM((1, CH), jnp.int32),
        pltpu.VMEM((1, CH), jnp.float32),
        pltpu.VMEM((CH, D), jnp.float32),
        pltpu.VMEM((CH, D), jnp.float32),
        pltpu.VMEM((RB, D), jnp.float32),
        pltpu.VMEM_SHARED((N, D), jnp.float32),
        pltpu.SemaphoreType.DMA,
        pltpu.SemaphoreType.DMA,
    ],
)
def _sc_aggregate(xall_ref, gidx_ref, dst_ref, comb_ref, cnt_ref, out_ref,
                  gb0, db0, cb0, cv0, gb1, db1, cb1, cv1, rows0, rows1,
                  zrows, agg_sh, sem0, sem1):
    c = lax.axis_index("c")
    s = lax.axis_index("s")
    wid = s * NC + c

    z16 = jnp.zeros((16,), jnp.float32)

    def zfill(i, carry):
        for k in range(D // 16):
            zrows[i, pl.ds(k * 16, 16)] = z16
        return carry

    lax.fori_loop(0, RB, zfill, 0)

    nz = jnp.where(s < EXTRA_R, BASE_R + 1, BASE_R)

    def zloop(t, carry):
        b = s + NS * t
        pltpu.sync_copy(zrows, agg_sh.at[pl.ds(b * RB, RB)])
        return carry

    lax.fori_loop(0, nz, zloop, 0)
    plsc.subcore_barrier()

    nw = jnp.where(wid < EXTRA_C, BASE_C + 1, BASE_C)

    def _start(t, gb, db, cb, cv, rows, sem):
        ch = wid + NW * t
        pltpu.sync_copy(gidx_ref.at[ch], gb)
        pltpu.sync_copy(dst_ref.at[ch], db)
        pltpu.sync_copy(comb_ref.at[ch], cb)
        pltpu.async_copy(cnt_ref.at[cb.at[0]], cv.at[0], sem)
        pltpu.async_copy(xall_ref.at[gb.at[0]], rows, sem)

    def _finish(gb, db, cb, cv, rows, sem):
        pltpu.make_async_copy(cnt_ref.at[cb.at[0]], cv.at[0], sem).wait()
        pltpu.make_async_copy(xall_ref.at[gb.at[0]], rows, sem).wait()

        def scale(i, carry):
            nv16 = 1.0 / jnp.maximum(cv[0, pl.ds(i * 16, 16)], 1.0)
            for el in range(16):
                nvv = _lane_bcast(nv16, el)
                m = i * 16 + el
                for k in range(D // 16):
                    rows[m, pl.ds(k * 16, 16)] = (
                        rows[m, pl.ds(k * 16, 16)] * nvv
                    )
            return carry

        lax.fori_loop(0, CH // 16, scale, 0)
        pltpu.sync_copy(rows, agg_sh.at[db.at[0]], add=True)

    _start(0, gb0, db0, cb0, cv0, rows0, sem0)

    def pair(p, carry):
        t1 = 2 * p + 1

        @pl.when(t1 < nw)
        def _():
            _start(t1, gb1, db1, cb1, cv1, rows1, sem1)

        _finish(gb0, db0, cb0, cv0, rows0, sem0)

        @pl.when(t1 < nw)
        def _():
            @pl.when(t1 + 1 < nw)
            def _():
                _start(t1 + 1, gb0, db0, cb0, cv0, rows0, sem0)

            _finish(gb1, db1, cb1, cv1, rows1, sem1)

        return carry

    lax.fori_loop(0, (nw + 1) // 2, pair, 0)
    plsc.subcore_barrier()

    def wloop(t, carry):
        b = s + NS * t
        pltpu.sync_copy(agg_sh.at[pl.ds(b * RB, RB)],
                        out_ref.at[pl.ds(c * N + b * RB, RB)])
        return carry

    lax.fori_loop(0, nz, wloop, 0)


NCH2 = E // CH
BASE_T, EXTRA_T = NCH2 // NW, NCH2 % NW


@functools.partial(
    pl.kernel,
    out_type=jax.ShapeDtypeStruct((NCH2, CH, D), jnp.float32),
    mesh=_mesh,
    scratch_types=[
        pltpu.VMEM((1, CH), jnp.int32),
        pltpu.VMEM((1, CH), jnp.int32),
        pltpu.VMEM((1, CH), jnp.int32),
        pltpu.VMEM((1, CH), jnp.int32),
        pltpu.VMEM((CH, D), jnp.float32),
        pltpu.VMEM((CH, D), jnp.float32),
        pltpu.VMEM((CH, D), jnp.float32),
        pltpu.VMEM((CH, D), jnp.float32),
        pltpu.SemaphoreType.DMA,
        pltpu.SemaphoreType.DMA,
    ],
)
def _sc_triplet(a_ref, b_ref, src_ref, dst_ref, out_ref,
                sb0, db0, sb1, db1, ar0, br0, ar1, br1, sem0, sem1):
    c = lax.axis_index("c")
    s = lax.axis_index("s")
    wid = s * NC + c
    nw = jnp.where(wid < EXTRA_T, BASE_T + 1, BASE_T)

    def _start(t, sb, db, ar, br, sem):
        ch = wid + NW * t
        pltpu.sync_copy(src_ref.at[ch], sb)
        pltpu.sync_copy(dst_ref.at[ch], db)
        pltpu.async_copy(a_ref.at[sb.at[0]], ar, sem)
        pltpu.async_copy(b_ref.at[db.at[0]], br, sem)

    def _finish(t, sb, db, ar, br, sem):
        pltpu.make_async_copy(a_ref.at[sb.at[0]], ar, sem).wait()
        pltpu.make_async_copy(b_ref.at[db.at[0]], br, sem).wait()

        def addl(i, carry):
            for el in range(16):
                m = i * 16 + el
                for k in range(D // 16):
                    plsc.addupdate(ar.at[m, pl.ds(k * 16, 16)],
                                   br[m, pl.ds(k * 16, 16)])
            return carry

        lax.fori_loop(0, CH // 16, addl, 0)
        ch = wid + NW * t
        pltpu.sync_copy(ar, out_ref.at[ch])

    _start(0, sb0, db0, ar0, br0, sem0)

    def pair(p, carry):
        t1 = 2 * p + 1

        @pl.when(t1 < nw)
        def _():
            _start(t1, sb1, db1, ar1, br1, sem1)

        _finish(2 * p, sb0, db0, ar0, br0, sem0)

        @pl.when(t1 < nw)
        def _():
            @pl.when(t1 + 1 < nw)
            def _():
                _start(t1 + 1, sb0, db0, ar0, br0, sem0)

            _finish(t1, sb1, db1, ar1, br1, sem1)

        return carry

    lax.fori_loop(0, (nw + 1) // 2, pair, 0)


def _transform_body(h_ref, w_ref, root_ref, b_ref, xall_ref, rootx_ref):
    r = pl.program_id(1)
    h = h_ref[...]
    xall_ref[0] = jnp.dot(h, w_ref[0], preferred_element_type=jnp.float32)

    @pl.when(r == 0)
    def _():
        rootx_ref[...] = (
            jnp.dot(h, root_ref[...], preferred_element_type=jnp.float32)
            + b_ref[...]
        )


def _tc_transform(h, W, root, b):
    return pl.pallas_call(
        _transform_body,
        grid=(NB, R),
        in_specs=[
            pl.BlockSpec((BN, D), lambda i, r: (i, 0)),
            pl.BlockSpec((1, D, D), lambda i, r: (r, 0, 0)),
            pl.BlockSpec((D, D), lambda i, r: (0, 0)),
            pl.BlockSpec((1, D), lambda i, r: (0, 0)),
        ],
        out_specs=[
            pl.BlockSpec((1, BN, D), lambda i, r: (r, i, 0)),
            pl.BlockSpec((BN, D), lambda i, r: (i, 0)),
        ],
        out_shape=[
            jax.ShapeDtypeStruct((R, N, D), jnp.float32),
            jax.ShapeDtypeStruct((N, D), jnp.float32),
        ],
    )(h, W, root, b.reshape(1, D))


def _transform2_body(aggp_ref, rootx_ref, w_ref, root_ref, b_ref,
                     xall_ref, rootx2_ref):
    r = pl.program_id(1)
    h = jax.nn.relu(aggp_ref[0] + aggp_ref[1] + rootx_ref[...])
    xall_ref[0] = jnp.dot(h, w_ref[0], preferred_element_type=jnp.float32)

    @pl.when(r == 0)
    def _():
        rootx2_ref[...] = (
            jnp.dot(h, root_ref[...], preferred_element_type=jnp.float32)
            + b_ref[...]
        )


def _tc_transform2(aggp, rootx, W, root, b):
    return pl.pallas_call(
        _transform2_body,
        grid=(NB, R),
        in_specs=[
            pl.BlockSpec((2, BN, D), lambda i, r: (0, i, 0)),
            pl.BlockSpec((BN, D), lambda i, r: (i, 0)),
            pl.BlockSpec((1, D, D), lambda i, r: (r, 0, 0)),
            pl.BlockSpec((D, D), lambda i, r: (0, 0)),
            pl.BlockSpec((1, D), lambda i, r: (0, 0)),
        ],
        out_specs=[
            pl.BlockSpec((1, BN, D), lambda i, r: (r, i, 0)),
            pl.BlockSpec((BN, D), lambda i, r: (i, 0)),
        ],
        out_shape=[
            jax.ShapeDtypeStruct((R, N, D), jnp.float32),
            jax.ShapeDtypeStruct((N, D), jnp.float32),
        ],
    )(aggp, rootx, W, root, b.reshape(1, D))


def _proj_body(aggp_ref, rootx_ref, wt_ref, wb_ref, pb_ref, a_ref, b_ref):
    h = jax.nn.relu(aggp_ref[0] + aggp_ref[1] + rootx_ref[...])
    a_ref[...] = (
        jnp.dot(h, wt_ref[...], preferred_element_type=jnp.float32) + pb_ref[...]
    )
    b_ref[...] = jnp.dot(h, wb_ref[...], preferred_element_type=jnp.float32)


def _tc_proj(aggp, rootx, projW, projb):
    return pl.pallas_call(
        _proj_body,
        grid=(NB,),
        in_specs=[
            pl.BlockSpec((2, BN, D), lambda i: (0, i, 0)),
            pl.BlockSpec((BN, D), lambda i: (i, 0)),
            pl.BlockSpec((D, D), lambda i: (0, 0)),
            pl.BlockSpec((D, D), lambda i: (0, 0)),
            pl.BlockSpec((1, D), lambda i: (0, 0)),
        ],
        out_specs=[
            pl.BlockSpec((BN, D), lambda i: (i, 0)),
            pl.BlockSpec((BN, D), lambda i: (i, 0)),
        ],
        out_shape=[
            jax.ShapeDtypeStruct((N, D), jnp.float32),
            jax.ShapeDtypeStruct((N, D), jnp.float32),
        ],
    )(aggp, rootx, projW[:D], projW[D:], projb.reshape(1, D))


def kernel(x, edge_index, edge_type, W1, root1, b1, W2, root2, b2, projW, projb):
    src, dst = edge_index[0], edge_index[1]
    gidx3 = (edge_type * N + src).reshape(NCH, 1, CH)
    comb3 = (dst * R + edge_type).reshape(G, 2, CH)
    src3t = src.reshape(NCH2, 1, CH)
    dst3t = dst.reshape(NCH2, 1, CH)
    dst3 = dst.reshape(NCH, 1, CH)

    comb_ch = comb3.reshape(NCH, 1, CH)
    cntv = _sc_counts(comb3)

    x_all1, rootx1 = _tc_transform(x, W1, root1, b1)
    aggp1 = _sc_aggregate(x_all1.reshape(R * N, D), gidx3, dst3, comb_ch, cntv)

    x_all2, rootx2 = _tc_transform2(
        aggp1.reshape(2, N, D), rootx1, W2, root2, b2)
    aggp2 = _sc_aggregate(x_all2.reshape(R * N, D), gidx3, dst3, comb_ch, cntv)

    A, B = _tc_proj(aggp2.reshape(2, N, D), rootx2, projW, projb)
    return _sc_triplet(A, B, src3t, dst3t).reshape(E, D)


# speedup vs baseline: 1.1355x; 1.1355x over previous
import functools

import jax
import jax.numpy as jnp
from jax import lax
from jax.experimental import pallas as pl
from jax.experimental.pallas import tpu as pltpu
from jax.experimental.pallas import tpu_sc as plsc

N = 10000
E = 160000
R = 16
D = 128

BN = 1000
NB = N // BN

NC = 2
NS = 16
NW = NC * NS
CH = 128
G = E // (2 * CH)
BASE_W, EXTRA_W = G // NW, G % NW
BASE_S, EXTRA_S = G // NS, G % NS
RB = 40
NRB = N // RB
BASE_R, EXTRA_R = NRB // NS, NRB % NS
NCH = E // CH
BASE_C, EXTRA_C = NCH // NW, NCH % NW
CZ = 1280
NCZ = (N * R) // CZ

_mesh = plsc.VectorSubcoreMesh(core_axis_name="c", subcore_axis_name="s")

_GDN = lax.GatherDimensionNumbers(
    offset_dims=(), collapsed_slice_dims=(0,), start_index_map=(0,)
)


def _lane_bcast(vec16, lane):
    idx = jnp.full((16,), lane, jnp.int32)
    return lax.gather(vec16, idx[:, None], _GDN, (1,),
                      mode=lax.GatherScatterMode.PROMISE_IN_BOUNDS)


@functools.partial(
    pl.kernel,
    out_type=jax.ShapeDtypeStruct((N * R,), jnp.float32),
    mesh=_mesh,
    scratch_types=[
        pltpu.VMEM((2, CH), jnp.int32),
        pltpu.VMEM((CH,), jnp.float32),
        pltpu.VMEM((CZ,), jnp.float32),
        pltpu.VMEM_SHARED((N * R,), jnp.float32),
    ],
)
def _sc_counts(comb_ref, cnt_ref, cb2, ones, zbuf, cnt_sh):
    c = lax.axis_index("c")
    s = lax.axis_index("s")

    z16 = jnp.zeros((16,), jnp.float32)
    o16 = jnp.ones((16,), jnp.float32)

    def zfill(i, carry):
        zbuf[pl.ds(i * 16, 16)] = z16
        return carry

    lax.fori_loop(0, CZ // 16, zfill, 0)
    for k in range(CH // 16):
        ones[pl.ds(k * 16, 16)] = o16

    @pl.when(c == 0)
    def _():
        nz = jnp.where(s < NCZ % NS, NCZ // NS + 1, NCZ // NS)

        def zloop(t, carry):
            b = s + NS * t
            pltpu.sync_copy(zbuf, cnt_sh.at[pl.ds(b * CZ, CZ)])
            return carry

        lax.fori_loop(0, nz, zloop, 0)

    plsc.subcore_barrier()

    @pl.when(c == 0)
    def _():
        nb = jnp.where(s < EXTRA_S, BASE_S + 1, BASE_S)

        def bloop(t, carry):
            g = s + NS * t
            pltpu.sync_copy(comb_ref.at[g], cb2)
            for j in range(2):
                pltpu.sync_copy(ones, cnt_sh.at[cb2.at[j]], add=True)
            return carry

        lax.fori_loop(0, nb, bloop, 0)

    plsc.subcore_barrier()

    @pl.when(c == 0)
    def _():
        nz = jnp.where(s < NCZ % NS, NCZ // NS + 1, NCZ // NS)

        def wloop(t, carry):
            b = s + NS * t
            pltpu.sync_copy(cnt_sh.at[pl.ds(b * CZ, CZ)],
                            cnt_ref.at[pl.ds(b * CZ, CZ)])
            return carry

        lax.fori_loop(0, nz, wloop, 0)


@functools.partial(
    pl.kernel,
    out_type=jax.ShapeDtypeStruct((NC * N, D), jnp.float32),
    mesh=_mesh,
    scratch_types=[
        pltpu.VMEM((3, CH), jnp.int32),
        pltpu.VMEM((1, CH), jnp.float32),
        pltpu.VMEM((3, CH), jnp.int32),
        pltpu.VMEM((1, CH), jnp.float32),
        pltpu.VMEM((CH, D), jnp.float32),
        pltpu.VMEM((CH, D), jnp.float32),
        pltpu.VMEM((RB, D), jnp.float32),
        pltpu.VMEM_SHARED((N, D), jnp.float32),
        pltpu.SemaphoreType.DMA,
        pltpu.SemaphoreType.DMA,
    ],
)
def _sc_aggregate(xall_ref, idx_ref, cnt_ref, out_ref,
                  ib0, cv0, ib1, cv1, rows0, rows1,
                  zrows, agg_sh, sem0, sem1):
    c = lax.axis_index("c")
    s = lax.axis_index("s")
    wid = s * NC + c

    z16 = jnp.zeros((16,), jnp.float32)

    def zfill(i, carry):
        for k in range(D // 16):
            zrows[i, pl.ds(k * 16, 16)] = z16
        return carry

    lax.fori_loop(0, RB, zfill, 0)

    nz = jnp.where(s < EXTRA_R, BASE_R + 1, BASE_R)

    def zloop(t, carry):
        b = s + NS * t
        pltpu.sync_copy(zrows, agg_sh.at[pl.ds(b * RB, RB)])
        return carry

    lax.fori_loop(0, nz, zloop, 0)
    plsc.subcore_barrier()

    nw = jnp.where(wid < EXTRA_C, BASE_C + 1, BASE_C)

    def _start(t, ib, cv, rows, sem):
        ch = wid + NW * t
        pltpu.sync_copy(idx_ref.at[ch], ib)
        pltpu.async_copy(cnt_ref.at[ib.at[2]], cv.at[0], sem)
        pltpu.async_copy(xall_ref.at[ib.at[0]], rows, sem)

    def _finish(ib, cv, rows, sem):
        pltpu.make_async_copy(cnt_ref.at[ib.at[2]], cv.at[0], sem).wait()
        pltpu.make_async_copy(xall_ref.at[ib.at[0]], rows, sem).wait()

        def scale(i, carry):
            nv16 = 1.0 / jnp.maximum(cv[0, pl.ds(i * 16, 16)], 1.0)
            for el in range(16):
                nvv = _lane_bcast(nv16, el)
                m = i * 16 + el
                for k in range(D // 16):
                    rows[m, pl.ds(k * 16, 16)] = (
                        rows[m, pl.ds(k * 16, 16)] * nvv
                    )
            return carry

        lax.fori_loop(0, CH // 16, scale, 0)
        pltpu.sync_copy(rows, agg_sh.at[ib.at[1]], add=True)

    _start(0, ib0, cv0, rows0, sem0)

    def pair(p, carry):
        t1 = 2 * p + 1

        @pl.when(t1 < nw)
        def _():
            _start(t1, ib1, cv1, rows1, sem1)

        _finish(ib0, cv0, rows0, sem0)

        @pl.when(t1 < nw)
        def _():
            @pl.when(t1 + 1 < nw)
            def _():
                _start(t1 + 1, ib0, cv0, rows0, sem0)

            _finish(ib1, cv1, rows1, sem1)

        return carry

    lax.fori_loop(0, (nw + 1) // 2, pair, 0)
    plsc.subcore_barrier()

    def wloop(t, carry):
        b = s + NS * t
        pltpu.sync_copy(agg_sh.at[pl.ds(b * RB, RB)],
                        out_ref.at[pl.ds(c * N + b * RB, RB)])
        return carry

    lax.fori_loop(0, nz, wloop, 0)


NCH2 = E // CH
BASE_T, EXTRA_T = NCH2 // NW, NCH2 % NW


@functools.partial(
    pl.kernel,
    out_type=jax.ShapeDtypeStruct((NCH2, CH, D), jnp.float32),
    mesh=_mesh,
    scratch_types=[
        pltpu.VMEM((2, CH), jnp.int32),
        pltpu.VMEM((2, CH), jnp.int32),
        pltpu.VMEM((CH, D), jnp.float32),
        pltpu.VMEM((CH, D), jnp.float32),
        pltpu.VMEM((CH, D), jnp.float32),
        pltpu.VMEM((CH, D), jnp.float32),
        pltpu.SemaphoreType.DMA,
        pltpu.SemaphoreType.DMA,
    ],
)
def _sc_triplet(a_ref, b_ref, tidx_ref, out_ref,
                tb0, tb1, ar0, br0, ar1, br1, sem0, sem1):
    c = lax.axis_index("c")
    s = lax.axis_index("s")
    wid = s * NC + c
    nw = jnp.where(wid < EXTRA_T, BASE_T + 1, BASE_T)

    def _start(t, tb, ar, br, sem):
        ch = wid + NW * t
        pltpu.sync_copy(tidx_ref.at[ch], tb)
        pltpu.async_copy(a_ref.at[tb.at[0]], ar, sem)
        pltpu.async_copy(b_ref.at[tb.at[1]], br, sem)

    def _finish(t, tb, ar, br, sem):
        pltpu.make_async_copy(a_ref.at[tb.at[0]], ar, sem).wait()
        pltpu.make_async_copy(b_ref.at[tb.at[1]], br, sem).wait()

        def addl(i, carry):
            for el in range(16):
                m = i * 16 + el
                for k in range(D // 16):
                    plsc.addupdate(ar.at[m, pl.ds(k * 16, 16)],
                                   br[m, pl.ds(k * 16, 16)])
            return carry

        lax.fori_loop(0, CH // 16, addl, 0)
        ch = wid + NW * t
        pltpu.sync_copy(ar, out_ref.at[ch])

    _start(0, tb0, ar0, br0, sem0)

    def pair(p, carry):
        t1 = 2 * p + 1

        @pl.when(t1 < nw)
        def _():
            _start(t1, tb1, ar1, br1, sem1)

        _finish(2 * p, tb0, ar0, br0, sem0)

        @pl.when(t1 < nw)
        def _():
            @pl.when(t1 + 1 < nw)
            def _():
                _start(t1 + 1, tb0, ar0, br0, sem0)

            _finish(t1, tb1, ar1, br1, sem1)

        return carry

    lax.fori_loop(0, (nw + 1) // 2, pair, 0)


def _transform_body(h_ref, w_ref, root_ref, b_ref, xall_ref, rootx_ref):
    r = pl.program_id(1)
    h = h_ref[...]
    xall_ref[0] = jnp.dot(h, w_ref[0], preferred_element_type=jnp.float32)

    @pl.when(r == 0)
    def _():
        rootx_ref[...] = (
            jnp.dot(h, root_ref[...], preferred_element_type=jnp.float32)
            + b_ref[...]
        )


def _tc_transform(h, W, root, b):
    return pl.pallas_call(
        _transform_body,
        grid=(NB, R),
        in_specs=[
            pl.BlockSpec((BN, D), lambda i, r: (i, 0)),
            pl.BlockSpec((1, D, D), lambda i, r: (r, 0, 0)),
            pl.BlockSpec((D, D), lambda i, r: (0, 0)),
            pl.BlockSpec((1, D), lambda i, r: (0, 0)),
        ],
        out_specs=[
            pl.BlockSpec((1, BN, D), lambda i, r: (r, i, 0)),
            pl.BlockSpec((BN, D), lambda i, r: (i, 0)),
        ],
        out_shape=[
            jax.ShapeDtypeStruct((R, N, D), jnp.float32),
            jax.ShapeDtypeStruct((N, D), jnp.float32),
        ],
    )(h, W, root, b.reshape(1, D))


def _transform2_body(aggp_ref, rootx_ref, w_ref, root_ref, b_ref,
                     xall_ref, rootx2_ref):
    r = pl.program_id(1)
    h = jax.nn.relu(aggp_ref[0] + aggp_ref[1] + rootx_ref[...])
    xall_ref[0] = jnp.dot(h, w_ref[0], preferred_element_type=jnp.float32)

    @pl.when(r == 0)
    def _():
        rootx2_ref[...] = (
            jnp.dot(h, root_ref[...], preferred_element_type=jnp.float32)
            + b_ref[...]
        )


def _tc_transform2(aggp, rootx, W, root, b):
    return pl.pallas_call(
        _transform2_body,
        grid=(NB, R),
        in_specs=[
            pl.BlockSpec((2, BN, D), lambda i, r: (0, i, 0)),
            pl.BlockSpec((BN, D), lambda i, r: (i, 0)),
            pl.BlockSpec((1, D, D), lambda i, r: (r, 0, 0)),
            pl.BlockSpec((D, D), lambda i, r: (0, 0)),
            pl.BlockSpec((1, D), lambda i, r: (0, 0)),
        ],
        out_specs=[
            pl.BlockSpec((1, BN, D), lambda i, r: (r, i, 0)),
            pl.BlockSpec((BN, D), lambda i, r: (i, 0)),
        ],
        out_shape=[
            jax.ShapeDtypeStruct((R, N, D), jnp.float32),
            jax.ShapeDtypeStruct((N, D), jnp.float32),
        ],
    )(aggp, rootx, W, root, b.reshape(1, D))


def _proj_body(aggp_ref, rootx_ref, wt_ref, wb_ref, pb_ref, a_ref, b_ref):
    h = jax.nn.relu(aggp_ref[0] + aggp_ref[1] + rootx_ref[...])
    a_ref[...] = (
        jnp.dot(h, wt_ref[...], preferred_element_type=jnp.float32) + pb_ref[...]
    )
    b_ref[...] = jnp.dot(h, wb_ref[...], preferred_element_type=jnp.float32)


def _tc_proj(aggp, rootx, projW, projb):
    return pl.pallas_call(
        _proj_body,
        grid=(NB,),
        in_specs=[
            pl.BlockSpec((2, BN, D), lambda i: (0, i, 0)),
            pl.BlockSpec((BN, D), lambda i: (i, 0)),
            pl.BlockSpec((D, D), lambda i: (0, 0)),
            pl.BlockSpec((D, D), lambda i: (0, 0)),
            pl.BlockSpec((1, D), lambda i: (0, 0)),
        ],
        out_specs=[
            pl.BlockSpec((BN, D), lambda i: (i, 0)),
            pl.BlockSpec((BN, D), lambda i: (i, 0)),
        ],
        out_shape=[
            jax.ShapeDtypeStruct((N, D), jnp.float32),
            jax.ShapeDtypeStruct((N, D), jnp.float32),
        ],
    )(aggp, rootx, projW[:D], projW[D:], projb.reshape(1, D))


def kernel(x, edge_index, edge_type, W1, root1, b1, W2, root2, b2, projW, projb):
    src, dst = edge_index[0], edge_index[1]
    gidx = edge_type * N + src
    comb = dst * R + edge_type
    idx3 = jnp.stack(
        [gidx.reshape(NCH, CH), dst.reshape(NCH, CH), comb.reshape(NCH, CH)],
        axis=1)
    tidx3 = jnp.stack([src.reshape(NCH2, CH), dst.reshape(NCH2, CH)], axis=1)

    cntv = _sc_counts(comb.reshape(G, 2, CH))

    x_all1, rootx1 = _tc_transform(x, W1, root1, b1)
    aggp1 = _sc_aggregate(x_all1.reshape(R * N, D), idx3, cntv)

    x_all2, rootx2 = _tc_transform2(
        aggp1.reshape(2, N, D), rootx1, W2, root2, b2)
    aggp2 = _sc_aggregate(x_all2.reshape(R * N, D), idx3, cntv)

    A, B = _tc_proj(aggp2.reshape(2, N, D), rootx2, projW, projb)
    return _sc_triplet(A, B, tidx3).reshape(E, D)
